# Initial kernel scaffold; baseline (speedup 1.0000x reference)
import jax
import jax.numpy as jnp
from jax.experimental import pallas as pl


def kernel(x, w1, b1, w2, b2):
    raise NotImplementedError("write your pallas kernel here")


# speedup vs baseline: 4.0377x; 4.0377x over previous
import functools

import numpy as np
import jax
import jax.numpy as jnp
from jax import lax
from jax.experimental import pallas as pl
from jax.experimental.pallas import tpu as pltpu
from jax.experimental.pallas import tpu_sc as plsc

SEQ = 512
NFREQ = SEQ // 2
HID = NFREQ * 4
LAMBD = 0.01
TOPK = 8
BATCH = 128
CHAN = 64
ROWS = BATCH * CHAN

_n = np.arange(SEQ, dtype=np.float64)[:, None]
_k = np.arange(1, NFREQ + 1, dtype=np.float64)[None, :]
_ang = 2.0 * np.pi * _n * _k / SEQ
_CR = np.cos(_ang).astype(np.float32)
_CI = (-np.sin(_ang)).astype(np.float32)

_HP = dict(preferred_element_type=jnp.float32, precision=lax.Precision.HIGHEST)


_RB = 512
_GB = _RB // CHAN


def _dense_body(x_ref, cr_ref, ci_ref, w1a_ref, w1b_ref, w2a_ref, w2b_ref,
                b1a_ref, b1b_ref, b2a_ref, b2b_ref, out_ref):
    x = x_ref[...]
    xr = jnp.dot(x, cr_ref[...], **_HP)
    xi = jnp.dot(x, ci_ref[...], **_HP)
    dd = dict(preferred_element_type=jnp.float32)
    w1a = w1a_ref[...]
    w1b = w1b_ref[...]
    hr = jnp.maximum(jnp.dot(xr, w1a, **dd) - jnp.dot(xi, w1b, **dd)
                     + b1a_ref[...], 0.0)
    hi = jnp.maximum(jnp.dot(xi, w1a, **dd) + jnp.dot(xr, w1b, **dd)
                     + b1b_ref[...], 0.0)
    w2a = w2a_ref[...]
    w2b = w2b_ref[...]
    orr = jnp.dot(hr, w2a, **dd) - jnp.dot(hi, w2b, **dd) + b2a_ref[...]
    oii = jnp.dot(hi, w2a, **dd) + jnp.dot(hr, w2b, **dd) + b2b_ref[...]

    def softshrink(v):
        return jnp.where(v > LAMBD, v - LAMBD,
                         jnp.where(v < -LAMBD, v + LAMBD, 0.0))

    sr = softshrink(orr)
    si = softshrink(oii)
    mag = jnp.sqrt(sr * sr + si * si)
    parts = [
        jnp.sum(mag[g * CHAN:(g + 1) * CHAN, :], axis=0, keepdims=True)
        * (1.0 / CHAN)
        for g in range(_GB)
    ]
    out_ref[...] = jnp.concatenate(parts, axis=0)


def _dense_weights(xt, cr, ci, w1a, w1b, w2a, w2b, b1a, b1b, b2a, b2b):
    n_steps = ROWS // _RB
    full = lambda shape: pl.BlockSpec(shape, lambda i: (0, 0))
    return pl.pallas_call(
        _dense_body,
        grid=(n_steps,),
        in_specs=[
            pl.BlockSpec((_RB, SEQ), lambda i: (i, 0)),
            full((SEQ, NFREQ)),
            full((SEQ, NFREQ)),
            full((NFREQ, HID)),
            full((NFREQ, HID)),
            full((HID, SEQ)),
            full((HID, SEQ)),
            full((1, HID)),
            full((1, HID)),
            full((1, SEQ)),
            full((1, SEQ)),
        ],
        out_specs=pl.BlockSpec((_GB, SEQ), lambda i: (i, 0)),
        out_shape=jax.ShapeDtypeStruct((BATCH, SEQ), jnp.float32),
        compiler_params=pltpu.CompilerParams(
            dimension_semantics=("parallel",)),
    )(xt, cr, ci, w1a, w1b, w2a, w2b, b1a, b1b, b2a, b2b)


_NC = 2
_NS = 16
_NW = _NC * _NS
_RPW = BATCH // _NW
_NCHUNK = SEQ // 16


def _topk_body(w_hbm, out_hbm, wrows, orows):
    wid = lax.axis_index("s") * _NC + lax.axis_index("c")
    base = wid * _RPW
    pltpu.sync_copy(w_hbm.at[pl.ds(base, _RPW)], wrows)
    lane = lax.broadcasted_iota(jnp.int32, (16,), 0)
    zeros = jnp.zeros((16,), jnp.float32)
    for r in range(_RPW):
        tk, ti = plsc.sort_key_val(wrows[r, pl.ds(0, 16)], lane)
        for c in range(1, _NCHUNK):
            ck, cv = plsc.sort_key_val(wrows[r, pl.ds(c * 16, 16)],
                                       lane + c * 16)
            rk = lax.rev(ck, (0,))
            rv = lax.rev(cv, (0,))
            take = tk >= rk
            mk = jnp.where(take, tk, rk)
            mv = jnp.where(take, ti, rv)
            tk, ti = plsc.sort_key_val(mk, mv)
        top = lane >= (16 - TOPK)
        m = jnp.max(tk)
        e = jnp.where(top, jnp.exp(tk - m), 0.0)
        g = e / jnp.sum(e)
        for c in range(_NCHUNK):
            orows[r, pl.ds(c * 16, 16)] = zeros
        ridx = lane * 0 + r
        plsc.store_scatter(orows, [ridx, ti], g, mask=top)
    pltpu.sync_copy(orows, out_hbm.at[pl.ds(base, _RPW)])


@functools.cache
def _topk_gates_fn():
    return functools.partial(
        pl.kernel,
        out_type=jax.ShapeDtypeStruct((BATCH, SEQ), jnp.float32),
        mesh=plsc.VectorSubcoreMesh(core_axis_name="c", subcore_axis_name="s",
                                    num_cores=_NC, num_subcores=_NS),
        scratch_types=[
            pltpu.VMEM((_RPW, SEQ), jnp.float32),
            pltpu.VMEM((_RPW, SEQ), jnp.float32),
        ],
        compiler_params=pltpu.CompilerParams(needs_layout_passes=False),
    )(_topk_body)


def kernel(x, w1, b1, w2, b2):
    xt = jnp.transpose(x, (0, 2, 1)).reshape(ROWS, SEQ)
    cr = jnp.asarray(_CR)
    ci = jnp.asarray(_CI)
    weights = _dense_weights(
        xt, cr, ci, w1[0], w1[1], w2[0], w2[1],
        b1[0].reshape(1, HID), b1[1].reshape(1, HID),
        b2[0].reshape(1, SEQ), b2[1].reshape(1, SEQ))
    return _topk_gates_fn()(weights)
